# baseline (speedup 1.0000x reference)
import functools

import jax
import jax.numpy as jnp
from jax import lax
from jax.experimental import pallas as pl
from jax.experimental.pallas import tpu as pltpu
from jax.experimental.pallas import tpu_sc as plsc

_H = _W = 256
_HW = _H * _W
_B = 4
_N = 500_000
_P = _B * _N
_MAP = _B * _HW
_ACC = 2 * _MAP
_MIN_D = 0.1
_MAX_D = 10.0

_NC, _NS = 2, 16
_NW = _NC * _NS
_CHUNK = 2000
_NCHUNK = _P // _CHUNK
_GROUPS = _CHUNK // 16
_ZSLICE = _ACC // _NS


_NJ = (_NCHUNK + _NW - 1) // _NW


def _sc_scatter_body(x_hbm, y_hbm, z_hbm, d_hbm, out_hbm,
                     x_v0, y_v0, z_v0, d_v0, x_v1, y_v1, z_v1, d_v1,
                     ia_0, ib_0, va_0, vb_0, ia_1, ib_1, va_1, vb_1,
                     buf_v, acc, sem_in, sem_sc):
    in_bufs = ((x_v0, y_v0, z_v0, d_v0), (x_v1, y_v1, z_v1, d_v1))
    st_bufs = ((ia_0, ib_0, va_0, vb_0), (ia_1, ib_1, va_1, vb_1))
    cidx = lax.axis_index("c")
    sidx = lax.axis_index("s")
    wid = sidx * _NC + cidx

    def _zero(i, c):
        buf_v[pl.ds(i * 16, 16)] = jnp.zeros((16,), jnp.float32)
        return c
    lax.fori_loop(0, _ZSLICE // 16, _zero, 0)
    pltpu.sync_copy(buf_v, acc.at[pl.ds(sidx * _ZSLICE, _ZSLICE)])
    plsc.subcore_barrier()

    lanes = jnp.arange(16, dtype=jnp.int32)

    def _in_copies(j, k):
        cid = j * _NW + wid
        sl = pl.ds(cid * _CHUNK, _CHUNK)
        xv, yv, zv, dv = in_bufs[k]
        return (
            pltpu.make_async_copy(x_hbm.at[sl], xv, sem_in.at[k]),
            pltpu.make_async_copy(y_hbm.at[sl], yv, sem_in.at[k]),
            pltpu.make_async_copy(z_hbm.at[sl], zv, sem_in.at[k]),
            pltpu.make_async_copy(d_hbm.at[sl], dv, sem_in.at[k]),
        )

    def _sc_copies(j, k):
        ia, ib, va, vb = st_bufs[k]
        return (pltpu.make_async_copy(va, acc.at[ia], sem_sc.at[k]),
                pltpu.make_async_copy(vb, acc.at[ib], sem_sc.at[k]))

    def _start_in(j, k):
        cid = j * _NW + wid
        @pl.when(cid < _NCHUNK)
        def _():
            for cp in _in_copies(j, k):
                cp.start()

    def _wait_in(j, k):
        cid = j * _NW + wid
        @pl.when(cid < _NCHUNK)
        def _():
            for cp in _in_copies(j, k):
                cp.wait()

    def _wait_sc(j, k):
        cid = j * _NW + wid
        @pl.when((j >= 0) & (cid < _NCHUNK))
        def _():
            a, bcp = _sc_copies(j, k)
            a.wait()
            bcp.wait()

    def _compute(j, k):
        cid = j * _NW + wid
        @pl.when(cid < _NCHUNK)
        def _():
            b_off = (cid // (_N // _CHUNK)) * _HW
            xv, yv, zv, dv = in_bufs[k]
            ia, ib, va, vb = st_bufs[k]

            def _group(g, c):
                gsl = pl.ds(g * 16, 16)
                x = xv[gsl]
                y = yv[gsl]
                z = zv[gsl]
                d = dv[gsl]
                t = 1.0 + jnp.exp(-d)
                zs = jnp.maximum(z, _MIN_D)
                q = 1.0 / (t * zs)
                w = zs * q
                r = t * q
                u = (x * r + 0.5) * 256.0
                v = (y * r + 0.5) * 256.0
                valid = ((z > _MIN_D) & (u >= 0.0) & (u < 256.0)
                         & (v >= 0.0) & (v < 256.0))
                ui = u.astype(jnp.int32)
                vi = v.astype(jnp.int32)
                ui = ui - jnp.where(ui.astype(jnp.float32) > u, 1, 0)
                vi = vi - jnp.where(vi.astype(jnp.float32) > v, 1, 0)
                ui = jnp.minimum(jnp.maximum(ui, 0), _W - 1)
                vi = jnp.minimum(jnp.maximum(vi, 0), _H - 1)
                fidx = b_off + vi * _W + ui
                ia[gsl] = fidx
                ib[gsl] = fidx + _MAP
                va[gsl] = jnp.where(valid, zs * w, 0.0)
                vb[gsl] = jnp.where(valid, w, 0.0)
                return c
            lax.fori_loop(0, _GROUPS, _group, 0)
            a, bcp = _sc_copies(j, k)
            a.start(add=True)
            bcp.start(add=True)

    _start_in(0, 0)

    def _pair(jj, c):
        for k in (0, 1):
            j = jj * 2 + k
            _start_in(j + 1, 1 - k)
            _wait_in(j, k)
            _wait_sc(j - 2, k)
            _compute(j, k)
        return c
    lax.fori_loop(0, _NJ // 2, _pair, 0)
    _wait_sc(_NJ - 2, 0)
    _wait_sc(_NJ - 1, 1)

    plsc.subcore_barrier()
    pltpu.sync_copy(acc.at[pl.ds(sidx * _ZSLICE, _ZSLICE)], buf_v)
    pltpu.sync_copy(
        buf_v, out_hbm.at[pl.ds(cidx * _ACC + sidx * _ZSLICE, _ZSLICE)])


@jax.jit
def _scatter(x_flat, y_flat, z_flat, dens_flat):
    mesh = plsc.VectorSubcoreMesh(
        core_axis_name="c", subcore_axis_name="s",
        num_cores=_NC, num_subcores=_NS)
    fn = pl.kernel(
        _sc_scatter_body,
        out_type=jax.ShapeDtypeStruct((_NC * _ACC,), jnp.float32),
        mesh=mesh,
        scratch_types=(
            [pltpu.VMEM((_CHUNK,), jnp.float32)] * 8
            + [pltpu.VMEM((_CHUNK,), jnp.int32),
               pltpu.VMEM((_CHUNK,), jnp.int32),
               pltpu.VMEM((_CHUNK,), jnp.float32),
               pltpu.VMEM((_CHUNK,), jnp.float32)] * 2
            + [pltpu.VMEM((_ZSLICE,), jnp.float32),
               pltpu.VMEM_SHARED((_ACC,), jnp.float32),
               pltpu.SemaphoreType.DMA((2,)),
               pltpu.SemaphoreType.DMA((2,))]
        ),
    )
    return fn(x_flat, y_flat, z_flat, dens_flat)


def _dense_body(img_ref, maps_ref, out_ref):
    img = img_ref[0]
    mean = (img[0] + img[1] + img[2]) * (1.0 / 3.0)
    dg = 1.0 / (1.0 + jnp.exp(-mean)) * (_MAX_D - _MIN_D) + _MIN_D

    zw = maps_ref[0, 0, 0] + maps_ref[1, 0, 0]
    wsum = maps_ref[0, 1, 0] + maps_ref[1, 1, 0]
    has_w = wsum > 0.0
    depth = jnp.where(has_w, zw / jnp.where(has_w, wsum, 1.0), 0.0)

    def _norm(dmap):
        valid = dmap > 0.0
        validf = valid.astype(jnp.float32)
        has_valid = jnp.any(valid)
        vmin = jnp.min(jnp.where(valid, dmap, jnp.inf))
        vmax = jnp.max(jnp.where(valid, dmap, -jnp.inf))
        mn = jnp.maximum(vmin, _MIN_D)
        mx = jnp.minimum(vmax, _MAX_D)
        mn = jnp.where(has_valid, mn, 0.0)
        mx = jnp.where(has_valid, mx, _MAX_D)
        return (dmap - mn) / (mx - mn + 1e-8) * validf

    t = _norm(dg)
    p = _norm(depth)
    vm = (p > 0.0).astype(jnp.float32) * (t > 0.0).astype(jnp.float32)
    vs = jnp.sum(vm)
    l1 = jnp.sum(jnp.abs(p * vm - t * vm))

    ri = lax.broadcasted_iota(jnp.int32, (_H, _W), 0)
    ci = lax.broadcasted_iota(jnp.int32, (_H, _W), 1)
    band = (jnp.abs(ri - ci) <= 5).astype(jnp.float32)

    def _pool(x):
        s = jnp.dot(band, x, preferred_element_type=jnp.float32)
        s = jnp.dot(s, band, preferred_element_type=jnp.float32)
        return s * (1.0 / 121.0)

    mu1 = _pool(p)
    mu2 = _pool(t)
    s11 = _pool(p * p) - mu1 * mu1
    s22 = _pool(t * t) - mu2 * mu2
    s12 = _pool(p * t) - mu1 * mu2
    c1 = 0.01 ** 2
    c2 = 0.03 ** 2
    ssim_map = ((2.0 * mu1 * mu2 + c1) * (2.0 * s12 + c2)
                / ((mu1 * mu1 + mu2 * mu2 + c1) * (s11 + s22 + c2)))
    ssim_sum = jnp.sum(ssim_map * vm)

    i = lax.broadcasted_iota(jnp.int32, (1, 1, 128), 2)
    out_ref[...] = jnp.where(
        i == 0, vs, jnp.where(i == 1, l1, jnp.where(i == 2, ssim_sum, 0.0)))


def _dense(images, maps5):
    return pl.pallas_call(
        _dense_body,
        grid=(_B,),
        in_specs=[
            pl.BlockSpec((1, 3, _H, _W), lambda b: (b, 0, 0, 0)),
            pl.BlockSpec((_NC, 2, 1, _H, _W), lambda b: (0, 0, b, 0, 0)),
        ],
        out_specs=pl.BlockSpec((1, 1, 128), lambda b: (b, 0, 0)),
        out_shape=jax.ShapeDtypeStruct((_B, 1, 128), jnp.float32),
    )(images, maps5)


def kernel(images, points, densities):
    pts = points.reshape(-1, 3)
    acc = _scatter(pts[:, 0], pts[:, 1], pts[:, 2], densities.reshape(-1))
    maps5 = acc.reshape(_NC, 2, _B, _H, _W)
    partials = _dense(images, maps5)[:, 0, :]
    vs = jnp.sum(partials[:, 0])
    l1 = jnp.sum(partials[:, 1]) / (vs + 1e-8)
    ssim_l = 1.0 - jnp.sum(partials[:, 2]) / (vs + 1e-8)
    total = jnp.minimum(0.8 * l1 + 0.2 * ssim_l, 1.0)
    return jnp.where(vs < 10.0, jnp.float32(0.0), total)


# speedup vs baseline: 1.6493x; 1.6493x over previous
import functools

import jax
import jax.numpy as jnp
from jax import lax
from jax.experimental import pallas as pl
from jax.experimental.pallas import tpu as pltpu
from jax.experimental.pallas import tpu_sc as plsc

_H = _W = 256
_HW = _H * _W
_B = 4
_N = 500_000
_P = _B * _N
_MIN_D = 0.1
_MAX_D = 10.0

_NC, _NS = 2, 16
_NW = _NC * _NS
_Q = 4
_CHUNK = 2000
_QSTEP = 124_000
_GROUPS = _CHUNK // 16


def _sc_scatter_body(x_hbm, y_hbm, z_hbm, d_hbm, zeros_hbm, out_hbm,
                     x_v0, y_v0, z_v0, d_v0, x_v1, y_v1, z_v1, d_v1,
                     lacc, sem_in):
    in_bufs = ((x_v0, y_v0, z_v0, d_v0), (x_v1, y_v1, z_v1, d_v1))
    cidx = lax.axis_index("c")
    sidx = lax.axis_index("s")
    wid = sidx * _NC + cidx
    img = wid // 8
    kind = (wid // 4) % 2
    quarter = wid % 4
    base0 = img * _N + quarter * _QSTEP
    trips = jnp.where(quarter == 3, 64, 62)

    pltpu.sync_copy(zeros_hbm, lacc)

    lanes = jnp.arange(16, dtype=jnp.int32)

    def _in_copies(j, k):
        sl = pl.ds(base0 + j * _CHUNK, _CHUNK)
        xv, yv, zv, dv = in_bufs[k]
        return (
            pltpu.make_async_copy(x_hbm.at[sl], xv, sem_in.at[k]),
            pltpu.make_async_copy(y_hbm.at[sl], yv, sem_in.at[k]),
            pltpu.make_async_copy(z_hbm.at[sl], zv, sem_in.at[k]),
            pltpu.make_async_copy(d_hbm.at[sl], dv, sem_in.at[k]),
        )

    def _start_in(j, k):
        @pl.when(j < trips)
        def _():
            for cp in _in_copies(j, k):
                cp.start()

    def _compute(j, k):
        xv, yv, zv, dv = in_bufs[k]

        def _group(g, c):
            gsl = pl.ds(g * 16, 16)
            x = xv[gsl]
            y = yv[gsl]
            z = zv[gsl]
            d = dv[gsl]
            t = 1.0 + jnp.exp(-d)
            zs = jnp.maximum(z, _MIN_D)
            q = 1.0 / (t * zs)
            w = zs * q
            r = t * q
            u = (x * r + 0.5) * 256.0
            v = (y * r + 0.5) * 256.0
            valid = ((z > _MIN_D) & (u >= 0.0) & (u < 256.0)
                     & (v >= 0.0) & (v < 256.0))
            ui = u.astype(jnp.int32)
            vi = v.astype(jnp.int32)
            ui = ui - jnp.where(ui.astype(jnp.float32) > u, 1, 0)
            vi = vi - jnp.where(vi.astype(jnp.float32) > v, 1, 0)
            ui = jnp.minimum(jnp.maximum(ui, 0), _W - 1)
            vi = jnp.minimum(jnp.maximum(vi, 0), _H - 1)
            val = jnp.where(kind == 0, zs * w, w)
            val = jnp.where(valid, val, 0.0)
            plsc.addupdate_scatter(lacc, [vi * _W + ui], val)
            return c
        lax.fori_loop(0, _GROUPS, _group, 0)

    _start_in(0, 0)

    def _pair(jj, c):
        for k in (0, 1):
            j = jj * 2 + k
            _start_in(j + 1, 1 - k)
            for cp in _in_copies(j, k):
                cp.wait()
            _compute(j, k)
        return c
    lax.fori_loop(0, trips // 2, _pair, 0)

    pltpu.sync_copy(lacc, out_hbm.at[pl.ds(wid * _HW, _HW)])


@jax.jit
def _scatter(x_flat, y_flat, z_flat, dens_flat, zeros_map):
    mesh = plsc.VectorSubcoreMesh(
        core_axis_name="c", subcore_axis_name="s",
        num_cores=_NC, num_subcores=_NS)
    fn = pl.kernel(
        _sc_scatter_body,
        out_type=jax.ShapeDtypeStruct((_NW * _HW,), jnp.float32),
        mesh=mesh,
        compiler_params=pltpu.CompilerParams(needs_layout_passes=False),
        scratch_types=(
            [pltpu.VMEM((_CHUNK,), jnp.float32)] * 8
            + [pltpu.VMEM((_HW,), jnp.float32),
               pltpu.SemaphoreType.DMA((2,))]
        ),
    )
    return fn(x_flat, y_flat, z_flat, dens_flat, zeros_map)


def _dense_body(img_ref, maps_ref, out_ref):
    img = img_ref[0]
    mean = (img[0] + img[1] + img[2]) * (1.0 / 3.0)
    dg = 1.0 / (1.0 + jnp.exp(-mean)) * (_MAX_D - _MIN_D) + _MIN_D

    m = maps_ref[0]
    zw = m[0, 0] + m[0, 1] + m[0, 2] + m[0, 3]
    wsum = m[1, 0] + m[1, 1] + m[1, 2] + m[1, 3]
    has_w = wsum > 0.0
    depth = jnp.where(has_w, zw / jnp.where(has_w, wsum, 1.0), 0.0)

    def _norm(dmap):
        valid = dmap > 0.0
        validf = valid.astype(jnp.float32)
        has_valid = jnp.any(valid)
        vmin = jnp.min(jnp.where(valid, dmap, jnp.inf))
        vmax = jnp.max(jnp.where(valid, dmap, -jnp.inf))
        mn = jnp.maximum(vmin, _MIN_D)
        mx = jnp.minimum(vmax, _MAX_D)
        mn = jnp.where(has_valid, mn, 0.0)
        mx = jnp.where(has_valid, mx, _MAX_D)
        return (dmap - mn) / (mx - mn + 1e-8) * validf

    t = _norm(dg)
    p = _norm(depth)
    vm = (p > 0.0).astype(jnp.float32) * (t > 0.0).astype(jnp.float32)
    vs = jnp.sum(vm)
    l1 = jnp.sum(jnp.abs(p * vm - t * vm))

    ri = lax.broadcasted_iota(jnp.int32, (_H, _W), 0)
    ci = lax.broadcasted_iota(jnp.int32, (_H, _W), 1)
    band = (jnp.abs(ri - ci) <= 5).astype(jnp.float32)

    def _pool(x):
        s = jnp.dot(band, x, preferred_element_type=jnp.float32)
        s = jnp.dot(s, band, preferred_element_type=jnp.float32)
        return s * (1.0 / 121.0)

    mu1 = _pool(p)
    mu2 = _pool(t)
    s11 = _pool(p * p) - mu1 * mu1
    s22 = _pool(t * t) - mu2 * mu2
    s12 = _pool(p * t) - mu1 * mu2
    c1 = 0.01 ** 2
    c2 = 0.03 ** 2
    ssim_map = ((2.0 * mu1 * mu2 + c1) * (2.0 * s12 + c2)
                / ((mu1 * mu1 + mu2 * mu2 + c1) * (s11 + s22 + c2)))
    ssim_sum = jnp.sum(ssim_map * vm)

    i = lax.broadcasted_iota(jnp.int32, (1, 1, 128), 2)
    out_ref[...] = jnp.where(
        i == 0, vs, jnp.where(i == 1, l1, jnp.where(i == 2, ssim_sum, 0.0)))


def _dense(images, maps6):
    return pl.pallas_call(
        _dense_body,
        grid=(_B,),
        in_specs=[
            pl.BlockSpec((1, 3, _H, _W), lambda b: (b, 0, 0, 0)),
            pl.BlockSpec((1, 2, _Q, _H, _W), lambda b: (b, 0, 0, 0, 0)),
        ],
        out_specs=pl.BlockSpec((1, 1, 128), lambda b: (b, 0, 0)),
        out_shape=jax.ShapeDtypeStruct((_B, 1, 128), jnp.float32),
    )(images, maps6)


def kernel(images, points, densities):
    pts = points.reshape(-1, 3)
    zeros_map = jnp.zeros((_HW,), jnp.float32)
    acc = _scatter(pts[:, 0], pts[:, 1], pts[:, 2],
                   densities.reshape(-1), zeros_map)
    maps6 = acc.reshape(_B, 2, _Q, _H, _W)
    partials = _dense(images, maps6)[:, 0, :]
    vs = jnp.sum(partials[:, 0])
    l1 = jnp.sum(partials[:, 1]) / (vs + 1e-8)
    ssim_l = 1.0 - jnp.sum(partials[:, 2]) / (vs + 1e-8)
    total = jnp.minimum(0.8 * l1 + 0.2 * ssim_l, 1.0)
    return jnp.where(vs < 10.0, jnp.float32(0.0), total)
